# baseline (speedup 1.0000x reference)
import functools

import jax
import jax.numpy as jnp
from jax import lax
from jax.experimental import pallas as pl
from jax.experimental.pallas import tpu as pltpu
from jax.experimental.pallas import tpu_sc as plsc

N = 10000
E = 320000
D = 128
L = 3
H = D // 2

NC = 2
NS = 16
NW = NC * NS
EP = E // NW
C = 80
NCHUNK = EP // C
RPT = 624
RTAIL = N - NS * RPT

BLK = 2000
GRID = N // BLK


def _make_prop():
    mesh = plsc.VectorSubcoreMesh(core_axis_name="c", subcore_axis_name="s")

    @functools.partial(
        pl.kernel,
        mesh=mesh,
        out_type=jax.ShapeDtypeStruct((NC, 2, N, H), jnp.float32),
        scratch_types=[
            pltpu.VMEM((EP,), jnp.int32),
            pltpu.VMEM((EP,), jnp.int32),
            pltpu.VMEM((EP,), jnp.float32),
            pltpu.VMEM((C,), jnp.int32),
            pltpu.VMEM((C, H), jnp.float32),
            pltpu.VMEM((C, H), jnp.float32),
            pltpu.VMEM((C, H), jnp.float32),
            pltpu.VMEM((C, H), jnp.float32),
            pltpu.VMEM_SHARED((N, H), jnp.float32),
            pltpu.VMEM_SHARED((N, H), jnp.float32),
            pltpu.SemaphoreType.DMA,
            pltpu.SemaphoreType.DMA,
        ],
        compiler_params=pltpu.CompilerParams(use_tc_tiling_on_sc=False),
    )
    def prop(hpk_hbm, src_hbm, dst_hbm, w_hbm, zero_hbm, out_hbm,
             src_v, dst_v, w_v, dst_c, pk0, hi0, pk1, hi1,
             acc_lo, acc_hi, g0, g1):
        pk = (pk0, pk1)
        hi = (hi0, hi1)
        gsem = (g0, g1)
        c = lax.axis_index("c")
        s = lax.axis_index("s")
        wid = c * NS + s
        for acc in (acc_lo, acc_hi):
            pltpu.sync_copy(zero_hbm.at[pl.ds(s * RPT, RPT)],
                            acc.at[pl.ds(s * RPT, RPT)])

        @pl.when(s == NS - 1)
        def _():
            for acc in (acc_lo, acc_hi):
                pltpu.sync_copy(zero_hbm.at[pl.ds(NS * RPT, RTAIL)],
                                acc.at[pl.ds(NS * RPT, RTAIL)])

        plsc.subcore_barrier()
        base = wid * EP
        pltpu.sync_copy(src_hbm.at[pl.ds(base, EP)], src_v)
        pltpu.sync_copy(dst_hbm.at[pl.ds(base, EP)], dst_v)
        pltpu.sync_copy(w_hbm.at[pl.ds(base, EP)], w_v)

        def start_gather(i, k):
            pltpu.async_copy(hpk_hbm.at[src_v.at[pl.ds(i * C, C)]],
                             pk[k], gsem[k])

        def wait_gather(k):
            pltpu.make_async_copy(hpk_hbm.at[src_v.at[pl.ds(0, C)]],
                                  pk[k], gsem[k]).wait()

        def process(i, k):
            pkv = pk[k]
            hiv = hi[k]

            def scale(g, carry2):
                wv = w_v[pl.ds(i * C + g * 16, 16)]
                for j in range(16):
                    wsc = wv[j]
                    e = g * 16 + j
                    for q in range(H // 16):
                        sl = pl.ds(q * 16, 16)
                        u = lax.bitcast_convert_type(pkv[e, sl], jnp.int32)
                        lo16 = lax.bitcast_convert_type(u << 16, jnp.float32)
                        hi16 = lax.bitcast_convert_type(
                            u & jnp.int32(-65536), jnp.float32)
                        pkv[e, sl] = lo16 * wsc
                        hiv[e, sl] = hi16 * wsc
                return carry2

            lax.fori_loop(0, C // 16, scale, 0)
            for g in range(C // 16):
                dst_c[pl.ds(g * 16, 16)] = dst_v[pl.ds(i * C + g * 16, 16)]
            pltpu.sync_copy(pkv, acc_lo.at[dst_c], add=True)
            pltpu.sync_copy(hiv, acc_hi.at[dst_c], add=True)

        start_gather(0, 0)
        start_gather(1, 1)

        def pair(o, carry):
            for k in range(2):
                i = 2 * o + k
                wait_gather(k)
                process(i, k)

                @pl.when(i + 2 < NCHUNK)
                def _():
                    start_gather(i + 2, k)
            return carry

        lax.fori_loop(0, NCHUNK // 2, pair, 0)
        if NCHUNK % 2:
            wait_gather(0)
            process(NCHUNK - 1, 0)
        plsc.subcore_barrier()
        for half, acc in enumerate((acc_lo, acc_hi)):
            pltpu.sync_copy(acc.at[pl.ds(s * RPT, RPT)],
                            out_hbm.at[c, half, pl.ds(s * RPT, RPT)])

        @pl.when(s == NS - 1)
        def _():
            for half, acc in enumerate((acc_lo, acc_hi)):
                pltpu.sync_copy(acc.at[pl.ds(NS * RPT, RTAIL)],
                                out_hbm.at[c, half, pl.ds(NS * RPT, RTAIL)])

    return prop


def _pack(x):
    a = lax.bitcast_convert_type(x[:, :H].astype(jnp.bfloat16), jnp.uint16)
    b = lax.bitcast_convert_type(x[:, H:].astype(jnp.bfloat16), jnp.uint16)
    packed = a.astype(jnp.uint32) | (b.astype(jnp.uint32) << 16)
    return lax.bitcast_convert_type(packed, jnp.float32)


def _pack_x(x):
    def body(x_ref, o_ref):
        o_ref[...] = _pack(x_ref[...])

    return pl.pallas_call(
        body,
        grid=(GRID,),
        in_specs=[pl.BlockSpec((BLK, D), lambda i: (i, 0))],
        out_specs=pl.BlockSpec((BLK, H), lambda i: (i, 0)),
        out_shape=jax.ShapeDtypeStruct((N, H), jnp.float32),
    )(x)


def _combine(part):
    def body(p_ref, o_ref, opk_ref):
        lo = p_ref[0, 0] + p_ref[1, 0]
        hi = p_ref[0, 1] + p_ref[1, 1]
        full = jnp.concatenate([lo, hi], axis=1)
        o_ref[...] = full
        opk_ref[...] = _pack(full)

    return pl.pallas_call(
        body,
        grid=(GRID,),
        in_specs=[pl.BlockSpec((NC, 2, BLK, H), lambda i: (0, 0, i, 0))],
        out_specs=[pl.BlockSpec((BLK, D), lambda i: (i, 0)),
                   pl.BlockSpec((BLK, H), lambda i: (i, 0))],
        out_shape=[jax.ShapeDtypeStruct((N, D), jnp.float32),
                   jax.ShapeDtypeStruct((N, H), jnp.float32)],
    )(part)


def _final(x, p1, p2, part3, static_weights, W_lawm, alpha):
    def body(al_ref, x_ref, p1_ref, p2_ref, p3_ref, sw_ref, wl_ref,
             o_ref, m_scr):
        @pl.when(pl.program_id(0) == 0)
        def _():
            e0 = jnp.exp(al_ref[0])
            e1 = jnp.exp(al_ref[1])
            e2 = jnp.exp(al_ref[2])
            e3 = jnp.exp(al_ref[3])
            inv = 1.0 / (e0 + e1 + e2 + e3)
            wl = wl_ref[...]
            wc0 = jnp.dot(sw_ref[0], wl, preferred_element_type=jnp.float32)
            wc1 = jnp.dot(sw_ref[1], wl, preferred_element_type=jnp.float32)
            wc2 = jnp.dot(sw_ref[2], wl, preferred_element_type=jnp.float32)
            q2 = jnp.dot(wc0, wc1, preferred_element_type=jnp.float32)
            q3 = jnp.dot(q2, wc2, preferred_element_type=jnp.float32)
            rr = lax.broadcasted_iota(jnp.int32, (D, D), 0)
            cc = lax.broadcasted_iota(jnp.int32, (D, D), 1)
            eye = jnp.where(rr == cc, 1.0, 0.0).astype(jnp.float32)
            m_scr[0] = eye * (e0 * inv)
            m_scr[1] = wc0 * (e1 * inv)
            m_scr[2] = q2 * (e2 * inv)
            m_scr[3] = q3 * (e3 * inv)

        p3 = jnp.concatenate([p3_ref[0, 0] + p3_ref[1, 0],
                              p3_ref[0, 1] + p3_ref[1, 1]], axis=1)
        o_ref[...] = (
            jnp.dot(x_ref[...], m_scr[0], preferred_element_type=jnp.float32)
            + jnp.dot(p1_ref[...], m_scr[1], preferred_element_type=jnp.float32)
            + jnp.dot(p2_ref[...], m_scr[2], preferred_element_type=jnp.float32)
            + jnp.dot(p3, m_scr[3], preferred_element_type=jnp.float32)
        )

    row = lambda i: (i, 0)
    fixed2 = lambda i: (0, 0)
    return pl.pallas_call(
        body,
        grid=(GRID,),
        in_specs=[
            pl.BlockSpec(memory_space=pltpu.SMEM),
            pl.BlockSpec((BLK, D), row),
            pl.BlockSpec((BLK, D), row),
            pl.BlockSpec((BLK, D), row),
            pl.BlockSpec((NC, 2, BLK, H), lambda i: (0, 0, i, 0)),
            pl.BlockSpec((L, D, D), lambda i: (0, 0, 0)),
            pl.BlockSpec((D, D), fixed2),
        ],
        out_specs=pl.BlockSpec((BLK, D), row),
        out_shape=jax.ShapeDtypeStruct((N, D), jnp.float32),
        scratch_shapes=[pltpu.VMEM((4, D, D), jnp.float32)],
    )(alpha, x, p1, p2, part3, static_weights, W_lawm)


def kernel(all_emb, W_lawm, static_weights, alpha, edge_index, edge_weight):
    src = edge_index[0]
    dst = edge_index[1]
    zero = jnp.zeros((N, H), jnp.float32)
    prop = _make_prop()
    x_pk = _pack_x(all_emb)
    part1 = prop(x_pk, src, dst, edge_weight, zero)
    p1, p1_pk = _combine(part1)
    part2 = prop(p1_pk, src, dst, edge_weight, zero)
    p2, p2_pk = _combine(part2)
    part3 = prop(p2_pk, src, dst, edge_weight, zero)
    return _final(all_emb, p1, p2, part3, static_weights, W_lawm, alpha)


# speedup vs baseline: 1.1572x; 1.1572x over previous
import functools

import jax
import jax.numpy as jnp
from jax import lax
from jax.experimental import pallas as pl
from jax.experimental.pallas import tpu as pltpu
from jax.experimental.pallas import tpu_sc as plsc

N = 10000
E = 320000
D = 128
L = 3

NC = 2
NS = 16
NW = NC * NS
EP = E // NW
C = 80
NCHUNK = EP // C
NPIPE = 120
RPT = 624
RTAIL = N - NS * RPT

BLK = 2000
GRID = N // BLK


def _make_prop():
    mesh = plsc.VectorSubcoreMesh(core_axis_name="c", subcore_axis_name="s")

    @functools.partial(
        pl.kernel,
        mesh=mesh,
        out_type=jax.ShapeDtypeStruct((NC, N, D), jnp.float32),
        scratch_types=[
            pltpu.VMEM((C, D), jnp.float32),
            pltpu.VMEM((C, D), jnp.float32),
            pltpu.VMEM((C, D), jnp.float32),
            pltpu.VMEM((C, D), jnp.float32),
            pltpu.VMEM((C,), jnp.int32),
            pltpu.VMEM((C,), jnp.int32),
            pltpu.VMEM((C,), jnp.int32),
            pltpu.VMEM((C,), jnp.int32),
            pltpu.VMEM((C,), jnp.int32),
            pltpu.VMEM((C,), jnp.int32),
            pltpu.VMEM((C,), jnp.int32),
            pltpu.VMEM((C,), jnp.int32),
            pltpu.VMEM((C,), jnp.float32),
            pltpu.VMEM((C,), jnp.float32),
            pltpu.VMEM((C,), jnp.float32),
            pltpu.VMEM((C,), jnp.float32),
            pltpu.VMEM_SHARED((N, D), jnp.float32),
            pltpu.SemaphoreType.DMA,
            pltpu.SemaphoreType.DMA,
            pltpu.SemaphoreType.DMA,
            pltpu.SemaphoreType.DMA,
            pltpu.SemaphoreType.DMA,
            pltpu.SemaphoreType.DMA,
            pltpu.SemaphoreType.DMA,
            pltpu.SemaphoreType.DMA,
            pltpu.SemaphoreType.DMA,
            pltpu.SemaphoreType.DMA,
            pltpu.SemaphoreType.DMA,
            pltpu.SemaphoreType.DMA,
            pltpu.SemaphoreType.DMA,
            pltpu.SemaphoreType.DMA,
            pltpu.SemaphoreType.DMA,
            pltpu.SemaphoreType.DMA,
        ],
    )
    def prop(h_hbm, src_hbm, dst_hbm, w_hbm, zero_hbm, out_hbm,
             r0, r1, r2, r3, sb0, sb1, sb2, sb3, db0, db1, db2, db3,
             wb0, wb1, wb2, wb3, acc_sh,
             g0, g1, g2, g3, s0, s1, s2, s3,
             i0, i1, i2, i3, d0, d1, d2, d3):
        rows = (r0, r1, r2, r3)
        srcb = (sb0, sb1, sb2, sb3)
        dstb = (db0, db1, db2, db3)
        wbuf = (wb0, wb1, wb2, wb3)
        gsem = (g0, g1, g2, g3)
        ssem = (s0, s1, s2, s3)
        isem = (i0, i1, i2, i3)
        dsem = (d0, d1, d2, d3)
        c = lax.axis_index("c")
        s = lax.axis_index("s")
        wid = c * NS + s
        pltpu.sync_copy(zero_hbm.at[pl.ds(s * RPT, RPT)],
                        acc_sh.at[pl.ds(s * RPT, RPT)])

        @pl.when(s == NS - 1)
        def _():
            pltpu.sync_copy(zero_hbm.at[pl.ds(NS * RPT, RTAIL)],
                            acc_sh.at[pl.ds(NS * RPT, RTAIL)])

        plsc.subcore_barrier()
        base = wid * EP

        def start_src(i, k):
            pltpu.async_copy(src_hbm.at[pl.ds(base + i * C, C)],
                             srcb[k], isem[k])

        def wait_src(k):
            pltpu.make_async_copy(src_hbm.at[pl.ds(0, C)],
                                  srcb[k], isem[k]).wait()

        def start_dstw(i, k):
            pltpu.async_copy(dst_hbm.at[pl.ds(base + i * C, C)],
                             dstb[k], dsem[k])
            pltpu.async_copy(w_hbm.at[pl.ds(base + i * C, C)],
                             wbuf[k], dsem[k])

        def wait_dstw(k):
            pltpu.make_async_copy(dst_hbm.at[pl.ds(0, C)],
                                  dstb[k], dsem[k]).wait()
            pltpu.make_async_copy(w_hbm.at[pl.ds(0, C)],
                                  wbuf[k], dsem[k]).wait()

        def start_gather(k):
            pltpu.async_copy(h_hbm.at[srcb[k]], rows[k], gsem[k])

        def wait_gather(k):
            pltpu.make_async_copy(h_hbm.at[srcb[k]], rows[k], gsem[k]).wait()

        def start_scatter(k):
            pltpu.async_copy(rows[k], acc_sh.at[dstb[k]], ssem[k], add=True)

        def wait_scatter(k):
            pltpu.make_async_copy(rows[k], acc_sh.at[dstb[k]],
                                  ssem[k]).wait()

        def scale(k):
            rv = rows[k]
            wv_ref = wbuf[k]

            def grp(g, carry2):
                wv = wv_ref[pl.ds(g * 16, 16)]
                for j in range(16):
                    wsc = wv[j]
                    e = g * 16 + j
                    for dd in range(D // 16):
                        sl = pl.ds(dd * 16, 16)
                        rv[e, sl] = rv[e, sl] * wsc
                return carry2

            lax.fori_loop(0, C // 16, grp, 0)

        for k in range(4):
            start_src(k, k)
        start_dstw(0, 0)
        start_dstw(1, 1)
        wait_src(0)
        start_gather(0)
        wait_src(1)
        start_gather(1)

        def quad(o, carry):
            for k in range(4):
                i = 4 * o + k
                kn = (k + 2) % 4
                wait_gather(k)

                @pl.when(i + 4 < NPIPE)
                def _():
                    start_src(i + 4, k)

                wait_dstw(k)
                scale(k)
                start_scatter(k)

                @pl.when(i >= 2)
                def _():
                    wait_scatter(kn)

                @pl.when(i + 2 < NPIPE)
                def _():
                    start_dstw(i + 2, kn)
                    wait_src(kn)
                    start_gather(kn)
            return carry

        lax.fori_loop(0, NPIPE // 4, quad, 0)
        wait_scatter(2)
        wait_scatter(3)
        def tail(i, carry):
            pltpu.sync_copy(src_hbm.at[pl.ds(base + i * C, C)], srcb[0])
            pltpu.sync_copy(dst_hbm.at[pl.ds(base + i * C, C)], dstb[0])
            pltpu.sync_copy(w_hbm.at[pl.ds(base + i * C, C)], wbuf[0])
            start_gather(0)
            wait_gather(0)
            scale(0)
            pltpu.sync_copy(rows[0], acc_sh.at[dstb[0]], add=True)
            return carry

        lax.fori_loop(NPIPE, NCHUNK, tail, 0)
        plsc.subcore_barrier()
        pltpu.sync_copy(acc_sh.at[pl.ds(s * RPT, RPT)],
                        out_hbm.at[c, pl.ds(s * RPT, RPT)])

        @pl.when(s == NS - 1)
        def _():
            pltpu.sync_copy(acc_sh.at[pl.ds(NS * RPT, RTAIL)],
                            out_hbm.at[c, pl.ds(NS * RPT, RTAIL)])

    return prop


def _combine(part):
    def body(p_ref, o_ref):
        o_ref[...] = p_ref[0] + p_ref[1]

    return pl.pallas_call(
        body,
        grid=(GRID,),
        in_specs=[pl.BlockSpec((NC, BLK, D), lambda i: (0, i, 0))],
        out_specs=pl.BlockSpec((BLK, D), lambda i: (i, 0)),
        out_shape=jax.ShapeDtypeStruct((N, D), jnp.float32),
    )(part)


def _final(x, p1, p2, part3, static_weights, W_lawm, alpha):
    def body(al_ref, x_ref, p1_ref, p2_ref, p3_ref, sw_ref, wl_ref,
             o_ref, m_scr):
        @pl.when(pl.program_id(0) == 0)
        def _():
            e0 = jnp.exp(al_ref[0])
            e1 = jnp.exp(al_ref[1])
            e2 = jnp.exp(al_ref[2])
            e3 = jnp.exp(al_ref[3])
            inv = 1.0 / (e0 + e1 + e2 + e3)
            wl = wl_ref[...]
            wc0 = jnp.dot(sw_ref[0], wl, preferred_element_type=jnp.float32)
            wc1 = jnp.dot(sw_ref[1], wl, preferred_element_type=jnp.float32)
            wc2 = jnp.dot(sw_ref[2], wl, preferred_element_type=jnp.float32)
            q2 = jnp.dot(wc0, wc1, preferred_element_type=jnp.float32)
            q3 = jnp.dot(q2, wc2, preferred_element_type=jnp.float32)
            rr = lax.broadcasted_iota(jnp.int32, (D, D), 0)
            cc = lax.broadcasted_iota(jnp.int32, (D, D), 1)
            eye = jnp.where(rr == cc, 1.0, 0.0).astype(jnp.float32)
            m_scr[0] = eye * (e0 * inv)
            m_scr[1] = wc0 * (e1 * inv)
            m_scr[2] = q2 * (e2 * inv)
            m_scr[3] = q3 * (e3 * inv)

        p3 = p3_ref[0] + p3_ref[1]
        o_ref[...] = (
            jnp.dot(x_ref[...], m_scr[0], preferred_element_type=jnp.float32)
            + jnp.dot(p1_ref[...], m_scr[1], preferred_element_type=jnp.float32)
            + jnp.dot(p2_ref[...], m_scr[2], preferred_element_type=jnp.float32)
            + jnp.dot(p3, m_scr[3], preferred_element_type=jnp.float32)
        )

    row = lambda i: (i, 0)
    fixed2 = lambda i: (0, 0)
    return pl.pallas_call(
        body,
        grid=(GRID,),
        in_specs=[
            pl.BlockSpec(memory_space=pltpu.SMEM),
            pl.BlockSpec((BLK, D), row),
            pl.BlockSpec((BLK, D), row),
            pl.BlockSpec((BLK, D), row),
            pl.BlockSpec((NC, BLK, D), lambda i: (0, i, 0)),
            pl.BlockSpec((L, D, D), lambda i: (0, 0, 0)),
            pl.BlockSpec((D, D), fixed2),
        ],
        out_specs=pl.BlockSpec((BLK, D), row),
        out_shape=jax.ShapeDtypeStruct((N, D), jnp.float32),
        scratch_shapes=[pltpu.VMEM((4, D, D), jnp.float32)],
    )(alpha, x, p1, p2, part3, static_weights, W_lawm)


def kernel(all_emb, W_lawm, static_weights, alpha, edge_index, edge_weight):
    src = edge_index[0]
    dst = edge_index[1]
    zero = jnp.zeros((N, D), jnp.float32)
    prop = _make_prop()
    part1 = prop(all_emb, src, dst, edge_weight, zero)
    p1 = _combine(part1)
    part2 = prop(p1, src, dst, edge_weight, zero)
    p2 = _combine(part2)
    part3 = prop(p2, src, dst, edge_weight, zero)
    return _final(all_emb, p1, p2, part3, static_weights, W_lawm, alpha)
